# baseline (speedup 1.0000x reference)
import functools

import jax
import jax.numpy as jnp
from jax import lax
from jax.experimental import pallas as pl
from jax.experimental.pallas import tpu as pltpu
from jax.experimental.pallas import tpu_sc as plsc

NN = 10000
NP = 10112
NE = 320000
NW = 32
EPW = NE // NW
NCH = 80
CH = 128
RPT = NP // 16
ZR = 79
NG = 64
BM = 1264
GRID = NP // BM


def _sc_mesh():
    return plsc.VectorSubcoreMesh(core_axis_name="c", subcore_axis_name="s")


_SC_PARAMS = pltpu.CompilerParams(use_tc_tiling_on_sc=False)


@functools.cache
def _degree_kernel():
    @functools.partial(
        pl.kernel,
        out_type=jax.ShapeDtypeStruct((2, NP, 16), jnp.float32),
        mesh=_sc_mesh(),
        compiler_params=_SC_PARAMS,
        scratch_types=[
            pltpu.VMEM((NCH, CH), jnp.int32),
            pltpu.VMEM((CH, 16), jnp.float32),
            pltpu.VMEM((ZR, 16), jnp.float32),
            pltpu.VMEM_SHARED((NP, 16), jnp.float32),
            pltpu.SemaphoreType.DMA,
        ],
    )
    def k(dstw, out, didx, ones_b, zbuf, acc, sem):
        cid = lax.axis_index("c")
        sid = lax.axis_index("s")
        wid = sid * 2 + cid

        def fill(r, carry):
            zbuf[r, :] = jnp.zeros((16,), jnp.float32)
            return carry

        lax.fori_loop(0, ZR, fill, 0)

        def fill1(r, carry):
            ones_b[r, :] = jnp.full((16,), 1.0, jnp.float32)
            return carry

        lax.fori_loop(0, CH, fill1, 0)

        for b in range(8):
            pltpu.sync_copy(zbuf, acc.at[pl.ds(sid * RPT + b * ZR, ZR)])
        pltpu.sync_copy(dstw.at[wid], didx)
        plsc.subcore_barrier()

        def chunk(c, carry):
            pltpu.async_copy(ones_b, acc.at[didx.at[c]], sem, add=True)
            return carry

        lax.fori_loop(0, NCH, chunk, 0)

        def drain(c, carry):
            pltpu.make_async_copy(ones_b, acc.at[didx.at[c]], sem).wait()
            return carry

        lax.fori_loop(0, NCH, drain, 0)
        plsc.subcore_barrier()
        pltpu.sync_copy(acc.at[pl.ds(sid * RPT, RPT)],
                        out.at[cid, pl.ds(sid * RPT, RPT)])

    return k


@functools.cache
def _agg_kernel(F):
    NPH = 1 if F <= 64 else 2
    PCH = NCH // NPH

    @functools.partial(
        pl.kernel,
        out_type=jax.ShapeDtypeStruct((2, NP, F), jnp.float32),
        mesh=_sc_mesh(),
        compiler_params=_SC_PARAMS,
        scratch_types=[
            pltpu.VMEM((PCH, CH), jnp.int32),
            pltpu.VMEM((PCH, CH), jnp.int32),
            pltpu.VMEM((2, CH, F), jnp.float32),
            pltpu.VMEM_SHARED((NP, F), jnp.float32),
            [pltpu.SemaphoreType.DMA] * 2,
        ],
    )
    def k(hs, srcw, dstw, out, sidx, didx, gbuf, acc, gsems):
        cid = lax.axis_index("c")
        sid = lax.axis_index("s")
        wid = sid * 2 + cid

        def fill(r, carry):
            for j in range(F // 16):
                gbuf[0, r, pl.ds(j * 16, 16)] = jnp.zeros((16,), jnp.float32)
            return carry

        lax.fori_loop(0, CH, fill, 0)

        for b in range(RPT // CH):
            pltpu.sync_copy(gbuf.at[0],
                            acc.at[pl.ds(sid * RPT + b * CH, CH)])
        rem = RPT - (RPT // CH) * CH
        if rem:
            pltpu.sync_copy(gbuf.at[0, pl.ds(0, rem)],
                            acc.at[pl.ds(sid * RPT + RPT - rem, rem)])

        for ph in range(NPH):
            pltpu.sync_copy(srcw.at[wid, pl.ds(ph * PCH, PCH)], sidx)
            pltpu.sync_copy(dstw.at[wid, pl.ds(ph * PCH, PCH)], didx)
            if ph == 0:
                plsc.subcore_barrier()
            def chunk(c, carry):
                pltpu.async_copy(hs.at[sidx.at[c]], gbuf.at[0],
                                 gsems[0]).wait()
                pltpu.sync_copy(gbuf.at[0], acc.at[didx.at[c]], add=True)
                return carry

            lax.fori_loop(0, PCH, chunk, 0)

        plsc.subcore_barrier()
        pltpu.sync_copy(acc.at[pl.ds(sid * RPT, RPT)],
                        out.at[cid, pl.ds(sid * RPT, RPT)])

    return k


def _row_spec(F):
    return pl.BlockSpec((BM, F), lambda i: (i, 0))


def _p_spec(F):
    return pl.BlockSpec((2, BM, F), lambda i: (0, i, 0))


def _full_spec(shape):
    nd = len(shape)
    return pl.BlockSpec(shape, lambda i: (0,) * nd)


def _pre1(pdeg, xp, W1):
    def body(p_ref, x_ref, w_ref, dinv_ref, s_ref):
        p = p_ref[...]
        dinv = lax.rsqrt(1.0 + p[0] + p[1])
        dinv_ref[...] = dinv
        h = jnp.dot(x_ref[...], w_ref[...], preferred_element_type=jnp.float32,
            precision=lax.Precision.HIGHEST)
        s_ref[...] = h * dinv[:, :1]

    return pl.pallas_call(
        body,
        grid=(GRID,),
        in_specs=[_p_spec(16), _row_spec(128), _full_spec((128, 64))],
        out_specs=[_row_spec(16), _row_spec(64)],
        out_shape=[jax.ShapeDtypeStruct((NP, 16), jnp.float32),
                   jax.ShapeDtypeStruct((NP, 64), jnp.float32)],
    )(pdeg, xp, W1)


def _post_scale(p, s, dinv16, b):
    F = s.shape[1]

    def body(p_ref, s_ref, dinv_ref, b_ref, o_ref):
        pv = p_ref[...]
        dinv = dinv_ref[...][:, :1]
        y = jnp.maximum((pv[0] + pv[1] + s_ref[...]) * dinv + b_ref[...], 0.0)
        o_ref[...] = y * dinv

    return pl.pallas_call(
        body,
        grid=(GRID,),
        in_specs=[_p_spec(F), _row_spec(F), _row_spec(16), _full_spec((1, F))],
        out_specs=_row_spec(F),
        out_shape=jax.ShapeDtypeStruct((NP, F), jnp.float32),
    )(p, s, dinv16, b)


def _post_mm_scale(p, s, dinv16, W, b):
    Fi = s.shape[1]
    Fo = W.shape[1]

    def body(p_ref, s_ref, dinv_ref, w_ref, b_ref, o_ref):
        pv = p_ref[...]
        dinv = dinv_ref[...][:, :1]
        a = (pv[0] + pv[1] + s_ref[...]) * dinv
        y = jnp.maximum(
            jnp.dot(a, w_ref[...], preferred_element_type=jnp.float32,
            precision=lax.Precision.HIGHEST)
            + b_ref[...], 0.0)
        o_ref[...] = y * dinv

    return pl.pallas_call(
        body,
        grid=(GRID,),
        in_specs=[_p_spec(Fi), _row_spec(Fi), _row_spec(16),
                  _full_spec((Fi, Fo)), _full_spec((1, Fo))],
        out_specs=_row_spec(Fo),
        out_shape=jax.ShapeDtypeStruct((NP, Fo), jnp.float32),
    )(p, s, dinv16, W, b)


def _post_mm_mm_scale(p, s, dinv16, W3, b3, W4):
    Fi = s.shape[1]
    Fm = W3.shape[1]
    Fo = W4.shape[1]

    def body(p_ref, s_ref, dinv_ref, w3_ref, b3_ref, w4_ref, o_ref):
        pv = p_ref[...]
        dinv = dinv_ref[...][:, :1]
        a = (pv[0] + pv[1] + s_ref[...]) * dinv
        y = jnp.maximum(
            jnp.dot(a, w3_ref[...], preferred_element_type=jnp.float32,
            precision=lax.Precision.HIGHEST)
            + b3_ref[...], 0.0)
        h = jnp.dot(y, w4_ref[...], preferred_element_type=jnp.float32,
            precision=lax.Precision.HIGHEST)
        o_ref[...] = h * dinv

    return pl.pallas_call(
        body,
        grid=(GRID,),
        in_specs=[_p_spec(Fi), _row_spec(Fi), _row_spec(16),
                  _full_spec((Fi, Fm)), _full_spec((1, Fm)),
                  _full_spec((Fm, Fo))],
        out_specs=_row_spec(Fo),
        out_shape=jax.ShapeDtypeStruct((NP, Fo), jnp.float32),
    )(p, s, dinv16, W3, b3, W4)


def _post_bias_mm_scale(p, s, dinv16, b4, W5):
    Fi = s.shape[1]
    Fo = W5.shape[1]

    def body(p_ref, s_ref, dinv_ref, b4_ref, w5_ref, o_ref):
        pv = p_ref[...]
        dinv = dinv_ref[...][:, :1]
        y = jnp.maximum((pv[0] + pv[1] + s_ref[...]) * dinv + b4_ref[...], 0.0)
        h = jnp.dot(y, w5_ref[...], preferred_element_type=jnp.float32,
            precision=lax.Precision.HIGHEST)
        o_ref[...] = h * dinv

    return pl.pallas_call(
        body,
        grid=(GRID,),
        in_specs=[_p_spec(Fi), _row_spec(Fi), _row_spec(16),
                  _full_spec((1, Fi)), _full_spec((Fi, Fo))],
        out_specs=_row_spec(Fo),
        out_shape=jax.ShapeDtypeStruct((NP, Fo), jnp.float32),
    )(p, s, dinv16, b4, W5)


def _pool(p, s, dinv16, b5, bcol16):
    def body(p_ref, s_ref, dinv_ref, b_ref, bcol_ref, zsum_ref, zmax_ref,
             csum_ref):
        i = pl.program_id(0)
        pv = p_ref[...]
        dinv = dinv_ref[...][:, :1]
        y5 = jnp.maximum((pv[0] + pv[1] + s_ref[...]) * dinv + b_ref[...], 0.0)
        bcol = bcol_ref[...][:, :1]
        iota = lax.broadcasted_iota(jnp.int32, (BM, NG), 1)
        onehot = jnp.where(bcol == iota, 1.0, 0.0)

        @pl.when(i == 0)
        def _init():
            zsum_ref[...] = jnp.zeros_like(zsum_ref)
            zmax_ref[...] = jnp.zeros_like(zmax_ref)
            csum_ref[...] = jnp.zeros_like(csum_ref)

        dn = (((0,), (0,)), ((), ()))
        zsum_ref[...] += lax.dot_general(onehot, y5, dn,
                                         preferred_element_type=jnp.float32,
            precision=lax.Precision.HIGHEST)
        csum_ref[...] += lax.dot_general(onehot,
                                         jnp.ones((BM, 8), jnp.float32), dn,
                                         preferred_element_type=jnp.float32,
            precision=lax.Precision.HIGHEST)
        for g in range(NG):
            mg = jnp.where(bcol == g, 1.0, 0.0)
            m = jnp.max(y5 * mg, axis=0, keepdims=True)
            zmax_ref[g:g + 1, :] = jnp.maximum(zmax_ref[g:g + 1, :], m)

    return pl.pallas_call(
        body,
        grid=(GRID,),
        in_specs=[_p_spec(64), _row_spec(64), _row_spec(16),
                  _full_spec((1, 64)), _row_spec(16)],
        out_specs=[_full_spec((NG, 64)), _full_spec((NG, 64)),
                   _full_spec((NG, 8))],
        out_shape=[jax.ShapeDtypeStruct((NG, 64), jnp.float32),
                   jax.ShapeDtypeStruct((NG, 64), jnp.float32),
                   jax.ShapeDtypeStruct((NG, 8), jnp.float32)],
    )(p, s, dinv16, b5, bcol16)


def _head(zsum, zmax, csum, w1a, w1b, b1, w2, b2):
    def body(zsum_ref, zmax_ref, csum_ref, w1a_ref, w1b_ref, b1_ref, w2_ref,
             b2_ref, o_ref):
        cnt = jnp.maximum(csum_ref[...][:, :1], 1.0)
        zmean = zsum_ref[...] / cnt
        h = jnp.maximum(
            jnp.dot(zmean, w1a_ref[...], preferred_element_type=jnp.float32,
            precision=lax.Precision.HIGHEST)
            + jnp.dot(zmax_ref[...], w1b_ref[...],
                      preferred_element_type=jnp.float32,
            precision=lax.Precision.HIGHEST)
            + b1_ref[...], 0.0)
        o_ref[...] = jnp.dot(h, w2_ref[...],
                             preferred_element_type=jnp.float32,
            precision=lax.Precision.HIGHEST) + b2_ref[...]

    return pl.pallas_call(
        body,
        out_shape=jax.ShapeDtypeStruct((NG, 10), jnp.float32),
    )(zsum, zmax, csum, w1a, w1b, b1, w2, b2)


def kernel(x, edge_index, batch, W1, b1, W2, b2, W3, b3, W4, b4, W5, b5,
           fc1_W, fc1_b, fc2_W, fc2_b):
    src = edge_index[0].astype(jnp.int32)
    dst = edge_index[1].astype(jnp.int32)
    pad = NCH * CH - EPW
    srcw = jnp.concatenate(
        [src.reshape(NW, EPW), jnp.zeros((NW, pad), jnp.int32)],
        axis=1).reshape(NW, NCH, CH)
    dstw = jnp.concatenate(
        [dst.reshape(NW, EPW), jnp.full((NW, pad), NN, jnp.int32)],
        axis=1).reshape(NW, NCH, CH)
    xp = jnp.concatenate(
        [x, jnp.zeros((NP - NN, x.shape[1]), jnp.float32)], axis=0)
    bcol = jnp.concatenate(
        [batch.astype(jnp.int32), jnp.full((NP - NN,), NG, jnp.int32)])
    bcol16 = jnp.broadcast_to(bcol[:, None], (NP, 16))

    pdeg = _degree_kernel()(dstw)
    dinv16, s1 = _pre1(pdeg, xp, W1)
    p1 = _agg_kernel(64)(s1, srcw, dstw)
    s2 = _post_scale(p1, s1, dinv16, b1.reshape(1, 64))
    p2 = _agg_kernel(64)(s2, srcw, dstw)
    s3 = _post_mm_scale(p2, s2, dinv16, W2, b2.reshape(1, 128))
    p3 = _agg_kernel(128)(s3, srcw, dstw)
    s4 = _post_mm_mm_scale(p3, s3, dinv16, W3, b3.reshape(1, 256), W4)
    p4 = _agg_kernel(128)(s4, srcw, dstw)
    s5 = _post_bias_mm_scale(p4, s4, dinv16, b4.reshape(1, 128), W5)
    p5 = _agg_kernel(64)(s5, srcw, dstw)
    zsum, zmax, csum = _pool(p5, s5, dinv16, b5.reshape(1, 64), bcol16)
    return _head(zsum, zmax, csum, fc1_W[:64], fc1_W[64:],
                 fc1_b.reshape(1, 64), fc2_W, fc2_b.reshape(1, 10))


# speedup vs baseline: 1.3868x; 1.3868x over previous
import functools

import jax
import jax.numpy as jnp
from jax import lax
from jax.experimental import pallas as pl
from jax.experimental.pallas import tpu as pltpu
from jax.experimental.pallas import tpu_sc as plsc

NN = 10000
NP = 10112
NE = 320000
NW = 32
EPW = NE // NW
NCH = 79
CH = 128
RPT = NP // 16
ZR = 79
NG = 64
BM = 1264
GRID = NP // BM


def _sc_mesh():
    return plsc.VectorSubcoreMesh(core_axis_name="c", subcore_axis_name="s")


_SC_PARAMS = pltpu.CompilerParams(use_tc_tiling_on_sc=False)


@functools.cache
def _degree_kernel():
    @functools.partial(
        pl.kernel,
        out_type=jax.ShapeDtypeStruct((2, NP, 16), jnp.float32),
        mesh=_sc_mesh(),
        compiler_params=_SC_PARAMS,
        scratch_types=[
            pltpu.VMEM((NCH, CH), jnp.int32),
            pltpu.VMEM((CH, 16), jnp.float32),
            pltpu.VMEM((ZR, 16), jnp.float32),
            pltpu.VMEM_SHARED((NP, 16), jnp.float32),
        ],
    )
    def k(dstw, out, didx, ones_b, zbuf, acc):
        cid = lax.axis_index("c")
        sid = lax.axis_index("s")
        wid = sid * 2 + cid

        def fill(r, carry):
            zbuf[r, :] = jnp.zeros((16,), jnp.float32)
            return carry

        lax.fori_loop(0, ZR, fill, 0)

        def fill1(r, carry):
            ones_b[r, :] = jnp.full((16,), 1.0, jnp.float32)
            return carry

        lax.fori_loop(0, CH, fill1, 0)

        for b in range(8):
            pltpu.sync_copy(zbuf, acc.at[pl.ds(sid * RPT + b * ZR, ZR)])
        pltpu.sync_copy(dstw.at[wid], didx)
        plsc.subcore_barrier()

        def chunk(c, carry):
            pltpu.sync_copy(ones_b, acc.at[didx.at[c]], add=True)
            return carry

        lax.fori_loop(0, NCH, chunk, 0)
        plsc.subcore_barrier()
        pltpu.sync_copy(acc.at[pl.ds(sid * RPT, RPT)],
                        out.at[cid, pl.ds(sid * RPT, RPT)])

    return k


@functools.cache
def _agg_kernel(F):
    @functools.partial(
        pl.kernel,
        out_type=jax.ShapeDtypeStruct((2, NP, F), jnp.float32),
        mesh=_sc_mesh(),
        compiler_params=_SC_PARAMS,
        scratch_types=[
            pltpu.VMEM((NCH, CH), jnp.int32),
            pltpu.VMEM((NCH, CH), jnp.int32),
            pltpu.VMEM((CH, F), jnp.float32),
            pltpu.VMEM((ZR, F), jnp.float32),
            pltpu.VMEM_SHARED((NP, F), jnp.float32),
            pltpu.SemaphoreType.DMA,
        ],
    )
    def k(hs, srcw, dstw, out, sidx, didx, gbuf, zbuf, acc, sem):
        cid = lax.axis_index("c")
        sid = lax.axis_index("s")
        wid = sid * 2 + cid

        def fill(r, carry):
            for j in range(F // 16):
                zbuf[r, pl.ds(j * 16, 16)] = jnp.zeros((16,), jnp.float32)
            return carry

        lax.fori_loop(0, ZR, fill, 0)

        for b in range(8):
            pltpu.sync_copy(zbuf, acc.at[pl.ds(sid * RPT + b * ZR, ZR)])
        pltpu.sync_copy(srcw.at[wid], sidx)
        pltpu.sync_copy(dstw.at[wid], didx)
        plsc.subcore_barrier()

        def chunk(c, carry):
            pltpu.async_copy(hs.at[sidx.at[c]], gbuf, sem).wait()
            pltpu.sync_copy(gbuf, acc.at[didx.at[c]], add=True)
            return carry

        lax.fori_loop(0, NCH, chunk, 0)
        plsc.subcore_barrier()
        pltpu.sync_copy(acc.at[pl.ds(sid * RPT, RPT)],
                        out.at[cid, pl.ds(sid * RPT, RPT)])

    return k


def _row_spec(F):
    return pl.BlockSpec((BM, F), lambda i: (i, 0))


def _p_spec(F):
    return pl.BlockSpec((2, BM, F), lambda i: (0, i, 0))


def _full_spec(shape):
    nd = len(shape)
    return pl.BlockSpec(shape, lambda i: (0,) * nd)


def _pre1(pdeg, xp, W1):
    def body(p_ref, x_ref, w_ref, dinv_ref, s_ref):
        p = p_ref[...]
        dinv = lax.rsqrt(1.0 + p[0] + p[1])
        dinv_ref[...] = dinv
        h = jnp.dot(x_ref[...], w_ref[...], preferred_element_type=jnp.float32,
            precision=lax.Precision.HIGHEST)
        s_ref[...] = h * dinv[:, :1]

    return pl.pallas_call(
        body,
        grid=(GRID,),
        in_specs=[_p_spec(16), _row_spec(128), _full_spec((128, 64))],
        out_specs=[_row_spec(16), _row_spec(64)],
        out_shape=[jax.ShapeDtypeStruct((NP, 16), jnp.float32),
                   jax.ShapeDtypeStruct((NP, 64), jnp.float32)],
    )(pdeg, xp, W1)


def _post_scale(p, s, dinv16, b):
    F = s.shape[1]

    def body(p_ref, s_ref, dinv_ref, b_ref, o_ref):
        pv = p_ref[...]
        dinv = dinv_ref[...][:, :1]
        y = jnp.maximum((pv[0] + pv[1] + s_ref[...]) * dinv + b_ref[...], 0.0)
        o_ref[...] = y * dinv

    return pl.pallas_call(
        body,
        grid=(GRID,),
        in_specs=[_p_spec(F), _row_spec(F), _row_spec(16), _full_spec((1, F))],
        out_specs=_row_spec(F),
        out_shape=jax.ShapeDtypeStruct((NP, F), jnp.float32),
    )(p, s, dinv16, b)


def _post_mm_scale(p, s, dinv16, W, b):
    Fi = s.shape[1]
    Fo = W.shape[1]

    def body(p_ref, s_ref, dinv_ref, w_ref, b_ref, o_ref):
        pv = p_ref[...]
        dinv = dinv_ref[...][:, :1]
        a = (pv[0] + pv[1] + s_ref[...]) * dinv
        y = jnp.maximum(
            jnp.dot(a, w_ref[...], preferred_element_type=jnp.float32,
            precision=lax.Precision.HIGHEST)
            + b_ref[...], 0.0)
        o_ref[...] = y * dinv

    return pl.pallas_call(
        body,
        grid=(GRID,),
        in_specs=[_p_spec(Fi), _row_spec(Fi), _row_spec(16),
                  _full_spec((Fi, Fo)), _full_spec((1, Fo))],
        out_specs=_row_spec(Fo),
        out_shape=jax.ShapeDtypeStruct((NP, Fo), jnp.float32),
    )(p, s, dinv16, W, b)


def _post_mm_mm_scale(p, s, dinv16, W3, b3, W4):
    Fi = s.shape[1]
    Fm = W3.shape[1]
    Fo = W4.shape[1]

    def body(p_ref, s_ref, dinv_ref, w3_ref, b3_ref, w4_ref, o_ref):
        pv = p_ref[...]
        dinv = dinv_ref[...][:, :1]
        a = (pv[0] + pv[1] + s_ref[...]) * dinv
        y = jnp.maximum(
            jnp.dot(a, w3_ref[...], preferred_element_type=jnp.float32,
            precision=lax.Precision.HIGHEST)
            + b3_ref[...], 0.0)
        h = jnp.dot(y, w4_ref[...], preferred_element_type=jnp.float32,
            precision=lax.Precision.HIGHEST)
        o_ref[...] = h * dinv

    return pl.pallas_call(
        body,
        grid=(GRID,),
        in_specs=[_p_spec(Fi), _row_spec(Fi), _row_spec(16),
                  _full_spec((Fi, Fm)), _full_spec((1, Fm)),
                  _full_spec((Fm, Fo))],
        out_specs=_row_spec(Fo),
        out_shape=jax.ShapeDtypeStruct((NP, Fo), jnp.float32),
    )(p, s, dinv16, W3, b3, W4)


def _post_bias_mm_scale(p, s, dinv16, b4, W5):
    Fi = s.shape[1]
    Fo = W5.shape[1]

    def body(p_ref, s_ref, dinv_ref, b4_ref, w5_ref, o_ref):
        pv = p_ref[...]
        dinv = dinv_ref[...][:, :1]
        y = jnp.maximum((pv[0] + pv[1] + s_ref[...]) * dinv + b4_ref[...], 0.0)
        h = jnp.dot(y, w5_ref[...], preferred_element_type=jnp.float32,
            precision=lax.Precision.HIGHEST)
        o_ref[...] = h * dinv

    return pl.pallas_call(
        body,
        grid=(GRID,),
        in_specs=[_p_spec(Fi), _row_spec(Fi), _row_spec(16),
                  _full_spec((1, Fi)), _full_spec((Fi, Fo))],
        out_specs=_row_spec(Fo),
        out_shape=jax.ShapeDtypeStruct((NP, Fo), jnp.float32),
    )(p, s, dinv16, b4, W5)


def _pool(p, s, dinv16, b5, bcol16):
    def body(p_ref, s_ref, dinv_ref, b_ref, bcol_ref, zsum_ref, zmax_ref,
             csum_ref):
        i = pl.program_id(0)
        pv = p_ref[...]
        dinv = dinv_ref[...][:, :1]
        y5 = jnp.maximum((pv[0] + pv[1] + s_ref[...]) * dinv + b_ref[...], 0.0)
        bcol = bcol_ref[...][:, :1]
        iota = lax.broadcasted_iota(jnp.int32, (BM, NG), 1)
        onehot = jnp.where(bcol == iota, 1.0, 0.0)

        @pl.when(i == 0)
        def _init():
            zsum_ref[...] = jnp.zeros_like(zsum_ref)
            zmax_ref[...] = jnp.zeros_like(zmax_ref)
            csum_ref[...] = jnp.zeros_like(csum_ref)

        dn = (((0,), (0,)), ((), ()))
        zsum_ref[...] += lax.dot_general(onehot, y5, dn,
                                         preferred_element_type=jnp.float32,
            precision=lax.Precision.HIGHEST)
        csum_ref[...] += lax.dot_general(onehot,
                                         jnp.ones((BM, 8), jnp.float32), dn,
                                         preferred_element_type=jnp.float32,
            precision=lax.Precision.HIGHEST)
        for g in range(NG):
            mg = jnp.where(bcol == g, 1.0, 0.0)
            m = jnp.max(y5 * mg, axis=0, keepdims=True)
            zmax_ref[g:g + 1, :] = jnp.maximum(zmax_ref[g:g + 1, :], m)

    return pl.pallas_call(
        body,
        grid=(GRID,),
        in_specs=[_p_spec(64), _row_spec(64), _row_spec(16),
                  _full_spec((1, 64)), _row_spec(16)],
        out_specs=[_full_spec((NG, 64)), _full_spec((NG, 64)),
                   _full_spec((NG, 8))],
        out_shape=[jax.ShapeDtypeStruct((NG, 64), jnp.float32),
                   jax.ShapeDtypeStruct((NG, 64), jnp.float32),
                   jax.ShapeDtypeStruct((NG, 8), jnp.float32)],
    )(p, s, dinv16, b5, bcol16)


def _head(zsum, zmax, csum, w1a, w1b, b1, w2, b2):
    def body(zsum_ref, zmax_ref, csum_ref, w1a_ref, w1b_ref, b1_ref, w2_ref,
             b2_ref, o_ref):
        cnt = jnp.maximum(csum_ref[...][:, :1], 1.0)
        zmean = zsum_ref[...] / cnt
        h = jnp.maximum(
            jnp.dot(zmean, w1a_ref[...], preferred_element_type=jnp.float32,
            precision=lax.Precision.HIGHEST)
            + jnp.dot(zmax_ref[...], w1b_ref[...],
                      preferred_element_type=jnp.float32,
            precision=lax.Precision.HIGHEST)
            + b1_ref[...], 0.0)
        o_ref[...] = jnp.dot(h, w2_ref[...],
                             preferred_element_type=jnp.float32,
            precision=lax.Precision.HIGHEST) + b2_ref[...]

    return pl.pallas_call(
        body,
        out_shape=jax.ShapeDtypeStruct((NG, 10), jnp.float32),
    )(zsum, zmax, csum, w1a, w1b, b1, w2, b2)


def kernel(x, edge_index, batch, W1, b1, W2, b2, W3, b3, W4, b4, W5, b5,
           fc1_W, fc1_b, fc2_W, fc2_b):
    src = edge_index[0].astype(jnp.int32)
    dst = edge_index[1].astype(jnp.int32)
    pad = NCH * CH - EPW
    srcw = jnp.concatenate(
        [src.reshape(NW, EPW), jnp.zeros((NW, pad), jnp.int32)],
        axis=1).reshape(NW, NCH, CH)
    dstw = jnp.concatenate(
        [dst.reshape(NW, EPW), jnp.full((NW, pad), NN, jnp.int32)],
        axis=1).reshape(NW, NCH, CH)
    xp = jnp.concatenate(
        [x, jnp.zeros((NP - NN, x.shape[1]), jnp.float32)], axis=0)
    bcol = jnp.concatenate(
        [batch.astype(jnp.int32), jnp.full((NP - NN,), NG, jnp.int32)])
    bcol16 = jnp.broadcast_to(bcol[:, None], (NP, 16))

    pdeg = _degree_kernel()(dstw)
    dinv16, s1 = _pre1(pdeg, xp, W1)
    p1 = _agg_kernel(64)(s1, srcw, dstw)
    s2 = _post_scale(p1, s1, dinv16, b1.reshape(1, 64))
    p2 = _agg_kernel(64)(s2, srcw, dstw)
    s3 = _post_mm_scale(p2, s2, dinv16, W2, b2.reshape(1, 128))
    p3 = _agg_kernel(128)(s3, srcw, dstw)
    s4 = _post_mm_mm_scale(p3, s3, dinv16, W3, b3.reshape(1, 256), W4)
    p4 = _agg_kernel(128)(s4, srcw, dstw)
    s5 = _post_bias_mm_scale(p4, s4, dinv16, b4.reshape(1, 128), W5)
    p5 = _agg_kernel(64)(s5, srcw, dstw)
    zsum, zmax, csum = _pool(p5, s5, dinv16, b5.reshape(1, 64), bcol16)
    return _head(zsum, zmax, csum, fc1_W[:64], fc1_W[64:],
                 fc1_b.reshape(1, 64), fc2_W, fc2_b.reshape(1, 10))
